# baseline (speedup 1.0000x reference)
import jax
import jax.numpy as jnp
from jax import lax
from jax.experimental import pallas as pl

T_TOK = 16384
D_IN = 256
H_DIM = 512
NSEG = 16
BLK = 2048


def _rne_split(a):
    hi = a.astype(jnp.bfloat16)
    lo = (a - hi.astype(jnp.float32)).astype(jnp.bfloat16)
    return hi, lo


def _dot3(ah, al, bh, bl):
    d = lambda u, v: jnp.dot(u, v, preferred_element_type=jnp.float32)
    return d(ah, bh) + (d(ah, bl) + d(al, bh))


def _layer_body(xh_ref, xl_ref, wh_ref, wl_ref, b_ref, o_ref):
    o_ref[...] = jnp.maximum(
        _dot3(xh_ref[...], xl_ref[...], wh_ref[...], wl_ref[...])
        + b_ref[...], 0.0)


def _layer(xh, xl, wh, wl, b):
    n, k = xh.shape
    m = wh.shape[1]
    return pl.pallas_call(
        _layer_body,
        grid=(n // BLK,),
        in_specs=[
            pl.BlockSpec((BLK, k), lambda i: (i, 0)),
            pl.BlockSpec((BLK, k), lambda i: (i, 0)),
            pl.BlockSpec((k, m), lambda i: (0, 0)),
            pl.BlockSpec((k, m), lambda i: (0, 0)),
            pl.BlockSpec((1, m), lambda i: (0, 0)),
        ],
        out_specs=pl.BlockSpec((BLK, m), lambda i: (i, 0)),
        out_shape=jax.ShapeDtypeStruct((n, m), jnp.float32),
    )(xh, xl, wh, wl, b.reshape(1, m))


def kernel(flat, cu_seqlens, W1, b1, W2, b2, W3, b3):
    n_tok = flat.shape[0]
    token_ids = jnp.arange(n_tok, dtype=cu_seqlens.dtype)
    seg = jnp.searchsorted(cu_seqlens, token_ids, side="right") - 1
    h1 = _layer(*_rne_split(flat), *_rne_split(W1), b1)
    h2 = _layer(*_rne_split(h1), *_rne_split(W2), b2)
    ch, cl = _rne_split(h2)
    w3h, w3l = _rne_split(W3)
    d = lambda u, v: jnp.dot(u, v, preferred_element_type=jnp.float32)
    c = d(ch, w3h) + (d(ch, w3l) + d(cl, w3h)) + b3
    return jax.ops.segment_sum(c, seg, num_segments=NSEG)


# speedup vs baseline: 1.0797x; 1.0797x over previous
import jax
import jax.numpy as jnp
from jax import lax
from jax.experimental import pallas as pl

T_TOK = 16384
D_IN = 256
H_DIM = 512
NSEG = 16
BLK = 2048


def _rne_split(a):
    hi = a.astype(jnp.bfloat16)
    lo = (a - hi.astype(jnp.float32)).astype(jnp.bfloat16)
    return hi, lo


def _dot3(ah, al, bh, bl):
    d = lambda u, v: jnp.dot(u, v, preferred_element_type=jnp.float32)
    return d(ah, bh) + (d(ah, bl) + d(al, bh))


def _layer_body(xh_ref, xl_ref, wh_ref, wl_ref, b_ref, o_ref):
    o_ref[...] = jnp.maximum(
        _dot3(xh_ref[...], xl_ref[...], wh_ref[...], wl_ref[...])
        + b_ref[...], 0.0)


def _layer1_body(xh_ref, xl_ref, wh_ref, wl_ref, b_ref, oh_ref, ol_ref):
    h1 = jnp.maximum(
        _dot3(xh_ref[...], xl_ref[...], wh_ref[...], wl_ref[...])
        + b_ref[...], 0.0)
    hh = h1.astype(jnp.bfloat16)
    oh_ref[...] = hh
    ol_ref[...] = (h1 - hh.astype(jnp.float32)).astype(jnp.bfloat16)


def _layer1(xh, xl, wh, wl, b):
    n, k = xh.shape
    m = wh.shape[1]
    return pl.pallas_call(
        _layer1_body,
        grid=(n // BLK,),
        in_specs=[
            pl.BlockSpec((BLK, k), lambda i: (i, 0)),
            pl.BlockSpec((BLK, k), lambda i: (i, 0)),
            pl.BlockSpec((k, m), lambda i: (0, 0)),
            pl.BlockSpec((k, m), lambda i: (0, 0)),
            pl.BlockSpec((1, m), lambda i: (0, 0)),
        ],
        out_specs=[
            pl.BlockSpec((BLK, m), lambda i: (i, 0)),
            pl.BlockSpec((BLK, m), lambda i: (i, 0)),
        ],
        out_shape=[
            jax.ShapeDtypeStruct((n, m), jnp.bfloat16),
            jax.ShapeDtypeStruct((n, m), jnp.bfloat16),
        ],
    )(xh, xl, wh, wl, b.reshape(1, m))


def _layer(xh, xl, wh, wl, b):
    n, k = xh.shape
    m = wh.shape[1]
    return pl.pallas_call(
        _layer_body,
        grid=(n // BLK,),
        in_specs=[
            pl.BlockSpec((BLK, k), lambda i: (i, 0)),
            pl.BlockSpec((BLK, k), lambda i: (i, 0)),
            pl.BlockSpec((k, m), lambda i: (0, 0)),
            pl.BlockSpec((k, m), lambda i: (0, 0)),
            pl.BlockSpec((1, m), lambda i: (0, 0)),
        ],
        out_specs=pl.BlockSpec((BLK, m), lambda i: (i, 0)),
        out_shape=jax.ShapeDtypeStruct((n, m), jnp.float32),
    )(xh, xl, wh, wl, b.reshape(1, m))


def kernel(flat, cu_seqlens, W1, b1, W2, b2, W3, b3):
    n_tok = flat.shape[0]
    token_ids = jnp.arange(n_tok, dtype=cu_seqlens.dtype)
    seg = jnp.searchsorted(cu_seqlens, token_ids, side="right") - 1
    h1h, h1l = _layer1(*_rne_split(flat), *_rne_split(W1), b1)
    h2 = _layer(h1h, h1l, *_rne_split(W2), b2)
    ch, cl = _rne_split(h2)
    w3h, w3l = _rne_split(W3)
    d = lambda u, v: jnp.dot(u, v, preferred_element_type=jnp.float32)
    c = d(ch, w3h) + (d(ch, w3l) + d(cl, w3h)) + b3
    return jax.ops.segment_sum(c, seg, num_segments=NSEG)
